# baseline (speedup 1.0000x reference)
import functools

import jax
import jax.numpy as jnp
from jax import lax
from jax.experimental import pallas as pl
from jax.experimental.pallas import tpu as pltpu
from jax.experimental.pallas import tpu_sc as plsc

N, C, H, W = 2, 256, 100, 152
R = 512
PH = PW = 7
SR = 2
SCALE = 0.125
HW = H * W
NHW = N * HW

NC, NS, L = 2, 16, 16
NW = NC * NS
CELLS = PH * PW
OUT_ROWS = R * CELLS
G = 16
GROUPS_PER_W = OUT_ROWS // (NW * G)
SLOTS = SR * SR * 4
GR = SLOTS * G
CW = C // 2

NB = 3
NI = 4

ROIS_PER_W = R // NW
RSZ = C * CELLS

PB = 118
PBLK = 128
NPACK = N * PB
TAIL = HW - PB * PBLK


def _mesh():
    return plsc.VectorSubcoreMesh(
        core_axis_name="c", subcore_axis_name="s", num_cores=NC, num_subcores=NS
    )


@functools.partial(
    pl.kernel,
    out_type=jax.ShapeDtypeStruct((NHW * CW,), jnp.int32),
    mesh=_mesh(),
    compiler_params=pltpu.CompilerParams(needs_layout_passes=False),
    scratch_types=[
        pltpu.VMEM((2, C, PBLK), jnp.float32),
        pltpu.VMEM((2 * PBLK * CW,), jnp.int32),
        pltpu.VMEM((C * TAIL,), jnp.float32),
        pltpu.SemaphoreType.DMA,
        pltpu.SemaphoreType.DMA,
    ],
)
def _pack_sc(x_hbm, xtail_hbm, tbl_hbm, in_v, out_v, tail_v, sem_i, sem_o):
    wid = lax.axis_index("s") * NC + lax.axis_index("c")
    nfull = NPACK // NW
    nrem = NPACK - nfull * NW
    ng = jnp.where(wid < nrem, nfull + 1, nfull)
    g0 = wid * nfull + jnp.minimum(wid, nrem)
    rowpos = lax.iota(jnp.int32, L) * CW

    def fetch(g, slot):
        b = lax.div(g, PB)
        yx0 = lax.rem(g, PB) * PBLK
        pltpu.async_copy(
            x_hbm.at[b, pl.ds(0, C), pl.ds(yx0, PBLK)], in_v.at[slot], sem_i)

    fetch(g0, 0)

    def body(gi, _):
        g = g0 + gi
        slot = lax.rem(gi, 2)
        so = lax.rem(gi, 2) * (PBLK * CW)
        pltpu.make_async_copy(
            x_hbm.at[0, pl.ds(0, C), pl.ds(0, PBLK)], in_v.at[slot],
            sem_i).wait()

        @pl.when(gi + 1 < ng)
        def _():
            fetch(g + 1, lax.rem(gi + 1, 2))

        @pl.when(gi >= 2)
        def _():
            pltpu.make_async_copy(tbl_hbm.at[pl.ds(0, PBLK * CW)],
                                  out_v.at[pl.ds(so, PBLK * CW)], sem_o).wait()

        def u_body(u, _):
            for j in range(C // 32):
                ca = 32 * j + u
                for p in range(PBLK // L):
                    a = in_v[slot, ca, pl.ds(p * L, L)]
                    b = in_v[slot, ca + 16, pl.ds(p * L, L)]
                    wds = plsc.bitcast(
                        plsc.pack(a, b, format=plsc.PackFormat.INTERLEAVED),
                        jnp.int32)
                    plsc.store_scatter(
                        out_v, [so + (p * L) * CW + rowpos + 16 * j + u], wds)
            return 0

        lax.fori_loop(0, 16, u_body, 0)

        b = lax.div(g, PB)
        row0 = b * HW + lax.rem(g, PB) * PBLK
        pltpu.async_copy(out_v.at[pl.ds(so, PBLK * CW)],
                         tbl_hbm.at[pl.ds(row0 * CW, PBLK * CW)], sem_o)
        return 0

    lax.fori_loop(0, ng, body, 0)
    for _ in range(2):
        pltpu.make_async_copy(tbl_hbm.at[pl.ds(0, PBLK * CW)],
                              out_v.at[pl.ds(0, PBLK * CW)], sem_o).wait()

    @pl.when(wid >= NW - N)
    def _():
        b = wid - (NW - N)
        pltpu.sync_copy(xtail_hbm.at[pl.ds(b * C * TAIL, C * TAIL)], tail_v)

        def u_body(u, _):
            for j in range(C // 32):
                ca = 32 * j + u
                for p in range(TAIL // L):
                    a = tail_v[pl.ds(ca * TAIL + p * L, L)]
                    bb = tail_v[pl.ds((ca + 16) * TAIL + p * L, L)]
                    wds = plsc.bitcast(
                        plsc.pack(a, bb, format=plsc.PackFormat.INTERLEAVED),
                        jnp.int32)
                    plsc.store_scatter(
                        out_v, [(p * L) * CW + rowpos + 16 * j + u], wds)
            return 0

        lax.fori_loop(0, 16, u_body, 0)
        row0 = b * HW + PB * PBLK
        pltpu.sync_copy(out_v.at[pl.ds(0, TAIL * CW)],
                        tbl_hbm.at[pl.ds(row0 * CW, TAIL * CW)])


@functools.partial(
    pl.kernel,
    out_type=jax.ShapeDtypeStruct((OUT_ROWS * C,), jnp.float32),
    mesh=_mesh(),
    compiler_params=pltpu.CompilerParams(needs_layout_passes=False),
    scratch_types=[
        pltpu.VMEM((R * 5,), jnp.float32),
        pltpu.VMEM((NI * GR,), jnp.int32),
        pltpu.VMEM((NI * GR,), jnp.float32),
        pltpu.VMEM((NB * GR, CW), jnp.int32),
        pltpu.VMEM((2 * RSZ,), jnp.float32),
        pltpu.SemaphoreType.DMA,
        pltpu.SemaphoreType.DMA,
    ],
)
def _roi_pool_sc(feat_hbm, rois_hbm, out_hbm, rois_v, idx_v, w_v, buf_v,
                 ostage_v, sem_g, sem_f):
    wid = lax.axis_index("s") * NC + lax.axis_index("c")
    pltpu.sync_copy(rois_hbm, rois_v)
    roi0 = wid * ROIS_PER_W

    def emit(g):
        si = lax.rem(g, NI) * GR
        sb = lax.rem(g, NB) * GR
        base = wid * (GROUPS_PER_W * G) + g * G
        orv = base + lax.iota(jnp.int32, L)
        r = lax.div(orv, CELLS)
        rem = lax.rem(orv, CELLS)
        ph = lax.div(rem, PW)
        pw = lax.rem(rem, PW)

        r5 = r * 5
        col = lambda c: plsc.load_gather(rois_v, [r5 + c])
        b_i = col(0).astype(jnp.int32)
        x1 = col(1) * SCALE - 0.5
        y1 = col(2) * SCALE - 0.5
        x2 = col(3) * SCALE - 0.5
        y2 = col(4) * SCALE - 0.5
        bw = jnp.maximum(x2 - x1, 1.0) * (1.0 / PW)
        bh = jnp.maximum(y2 - y1, 1.0) * (1.0 / PH)
        base_row = b_i * HW
        ph_f = ph.astype(jnp.float32)
        pw_f = pw.astype(jnp.float32)

        wy, ry = [], []
        for s in range(SR):
            ys = y1 + (ph_f + (0.5 + s) / SR) * bh
            vy = jnp.where((ys > -1.0) & (ys < float(H)), 0.5, 0.0)
            yc = jnp.clip(ys, 0.0, float(H - 1))
            y0i = yc.astype(jnp.int32)
            ly = yc - y0i.astype(jnp.float32)
            wy.append([(1.0 - ly) * vy, ly * vy])
            ry.append([y0i * W, jnp.minimum(y0i + 1, H - 1) * W])
        wx, rx = [], []
        for t in range(SR):
            xs = x1 + (pw_f + (0.5 + t) / SR) * bw
            vx = jnp.where((xs > -1.0) & (xs < float(W)), 0.5, 0.0)
            xc = jnp.clip(xs, 0.0, float(W - 1))
            x0i = xc.astype(jnp.int32)
            lx = xc - x0i.astype(jnp.float32)
            wx.append([(1.0 - lx) * vx, lx * vx])
            rx.append([x0i, jnp.minimum(x0i + 1, W - 1)])

        k = 0
        for s in range(SR):
            for t in range(SR):
                for i in range(2):
                    for j in range(2):
                        idx_v[pl.ds(si + k * L, L)] = (
                            base_row + ry[s][i] + rx[t][j])
                        w_v[pl.ds(si + k * L, L)] = wy[s][i] * wx[t][j]
                        k += 1

        h = GR // 2
        pltpu.async_copy(feat_hbm.at[idx_v.at[pl.ds(si, h)]],
                         buf_v.at[pl.ds(sb, h)], sem_g)
        pltpu.async_copy(feat_hbm.at[idx_v.at[pl.ds(si + h, h)]],
                         buf_v.at[pl.ds(sb + h, h)], sem_g)

    for g0 in range(NB):
        emit(g0)

    iota49 = lax.iota(jnp.int32, L) * CELLS

    def group_body(g, _):
        si = lax.rem(g, NI) * GR
        sb = lax.rem(g, NB) * GR
        base = wid * (GROUPS_PER_W * G) + g * G
        r_first = lax.div(base, CELLS)
        cell_first = base - r_first * CELLS
        r_last = lax.div(base + G - 1, CELLS)

        pltpu.make_async_copy(feat_hbm.at[pl.ds(0, GR)],
                              buf_v.at[pl.ds(sb, GR)], sem_g).wait()

        new_roi = (cell_first == 0) | (r_last > r_first)

        @pl.when(new_roi & (r_last >= roi0 + 2))
        def _():
            pltpu.make_async_copy(out_hbm.at[pl.ds(0, RSZ)],
                                  ostage_v.at[pl.ds(0, RSZ)], sem_f).wait()

        def o_body(o, _):
            orv = base + o
            r_o = lax.div(orv, CELLS)
            cell_o = orv - r_o * CELLS
            p_o = lax.rem(r_o, 2) * RSZ + cell_o

            def k_body(kk, accs):
                m = kk * L + o
                wv = plsc.load_gather(w_v, [lax.broadcast(si + m, (L,))])
                out = []
                for j in range(C // 32):
                    a, b = plsc.unpack(
                        plsc.bitcast(buf_v[sb + m, pl.ds(j * L, L)],
                                     jnp.bfloat16),
                        format=plsc.PackFormat.INTERLEAVED,
                        preferred_element_type=jnp.float32,
                    )
                    out.append(accs[2 * j] + wv * a)
                    out.append(accs[2 * j + 1] + wv * b)
                return tuple(out)

            accs = lax.fori_loop(
                0, SLOTS, k_body,
                tuple(jnp.zeros((L,), jnp.float32) for _ in range(C // L)),
            )
            for j in range(C // 32):
                plsc.store_scatter(
                    ostage_v, [p_o + (32 * j) * CELLS + iota49], accs[2 * j])
                plsc.store_scatter(
                    ostage_v, [p_o + (32 * j + 16) * CELLS + iota49],
                    accs[2 * j + 1])
            return 0

        lax.fori_loop(0, G, o_body, 0)

        @pl.when(cell_first >= CELLS - G)
        def _():
            pltpu.async_copy(
                ostage_v.at[pl.ds(lax.rem(r_first, 2) * RSZ, RSZ)],
                out_hbm.at[pl.ds(r_first * RSZ, RSZ)], sem_f)

        @pl.when(g + NB < GROUPS_PER_W)
        def _():
            emit(g + NB)

        return 0

    lax.fori_loop(0, GROUPS_PER_W, group_body, 0)
    for _ in range(2):
        pltpu.make_async_copy(out_hbm.at[pl.ds(0, RSZ)],
                              ostage_v.at[pl.ds(0, RSZ)], sem_f).wait()


def kernel(input, rois):
    x3 = input.reshape(N, C, HW)
    xtail = lax.slice(x3, (0, 0, PB * PBLK), (N, C, HW)).reshape(-1)
    tbl = _pack_sc(x3, xtail)
    out_flat = _roi_pool_sc(tbl.reshape(NHW, CW), rois.reshape(-1))
    return out_flat.reshape(R, C, PH, PW)


# speedup vs baseline: 1.3175x; 1.3175x over previous
import functools

import numpy as np

import jax
import jax.numpy as jnp
from jax import lax
from jax.experimental import pallas as pl
from jax.experimental.pallas import tpu as pltpu
from jax.experimental.pallas import tpu_sc as plsc

N, C, H, W = 2, 256, 100, 152
R = 512
PH = PW = 7
SR = 2
SCALE = 0.125

NC, NS, L = 2, 16, 16
NW = NC * NS
OUT_ROWS = R * PH * PW
G = 16
GROUPS_PER_W = OUT_ROWS // (NW * G)
SLOTS = SR * SR * 4
GR = SLOTS * G

NB = 3
NI = 4
NO = 2

def _mesh():
    return plsc.VectorSubcoreMesh(
        core_axis_name="c", subcore_axis_name="s", num_cores=NC, num_subcores=NS
    )


@functools.partial(
    pl.kernel,
    out_type=jax.ShapeDtypeStruct((OUT_ROWS * C,), jnp.float32),
    mesh=_mesh(),
    compiler_params=pltpu.CompilerParams(needs_layout_passes=False),
    scratch_types=[
        pltpu.VMEM((R * 5,), jnp.float32),
        pltpu.VMEM((NI * GR,), jnp.int32),
        pltpu.VMEM((NI * GR,), jnp.float32),
        pltpu.VMEM((NB * GR, C // 2), jnp.int32),
        pltpu.VMEM((NO * G * C,), jnp.float32),
        pltpu.SemaphoreType.DMA,
        pltpu.SemaphoreType.DMA,
    ],
)
def _roi_pool_sc(feat_hbm, rois_hbm, out_hbm, rois_v, idx_v, w_v, buf_v,
                 ostage_v, sem_g, sem_o):
    wid = lax.axis_index("s") * NC + lax.axis_index("c")
    pltpu.sync_copy(rois_hbm, rois_v)

    def emit(g):
        si = lax.rem(g, NI) * GR
        sb = lax.rem(g, NB) * GR
        base = wid * (GROUPS_PER_W * G) + g * G
        orv = base + lax.iota(jnp.int32, L)
        r = lax.div(orv, PH * PW)
        rem = lax.rem(orv, PH * PW)
        ph = lax.div(rem, PW)
        pw = lax.rem(rem, PW)

        r5 = r * 5
        col = lambda c: plsc.load_gather(rois_v, [r5 + c])
        b_i = col(0).astype(jnp.int32)
        x1 = col(1) * SCALE - 0.5
        y1 = col(2) * SCALE - 0.5
        x2 = col(3) * SCALE - 0.5
        y2 = col(4) * SCALE - 0.5
        bw = jnp.maximum(x2 - x1, 1.0) * (1.0 / PW)
        bh = jnp.maximum(y2 - y1, 1.0) * (1.0 / PH)
        base_row = b_i * (H * W)
        ph_f = ph.astype(jnp.float32)
        pw_f = pw.astype(jnp.float32)

        wy, ry = [], []
        for s in range(SR):
            ys = y1 + (ph_f + (0.5 + s) / SR) * bh
            vy = jnp.where((ys > -1.0) & (ys < float(H)), 0.5, 0.0)
            yc = jnp.clip(ys, 0.0, float(H - 1))
            y0i = yc.astype(jnp.int32)
            ly = yc - y0i.astype(jnp.float32)
            wy.append([(1.0 - ly) * vy, ly * vy])
            ry.append([y0i * W, jnp.minimum(y0i + 1, H - 1) * W])
        wx, rx = [], []
        for t in range(SR):
            xs = x1 + (pw_f + (0.5 + t) / SR) * bw
            vx = jnp.where((xs > -1.0) & (xs < float(W)), 0.5, 0.0)
            xc = jnp.clip(xs, 0.0, float(W - 1))
            x0i = xc.astype(jnp.int32)
            lx = xc - x0i.astype(jnp.float32)
            wx.append([(1.0 - lx) * vx, lx * vx])
            rx.append([x0i, jnp.minimum(x0i + 1, W - 1)])

        k = 0
        for s in range(SR):
            for t in range(SR):
                for i in range(2):
                    for j in range(2):
                        idx_v[pl.ds(si + k * L, L)] = (
                            base_row + ry[s][i] + rx[t][j])
                        w_v[pl.ds(si + k * L, L)] = wy[s][i] * wx[t][j]
                        k += 1

        h = GR // 2
        pltpu.async_copy(feat_hbm.at[idx_v.at[pl.ds(si, h)]],
                         buf_v.at[pl.ds(sb, h)], sem_g)
        pltpu.async_copy(feat_hbm.at[idx_v.at[pl.ds(si + h, h)]],
                         buf_v.at[pl.ds(sb + h, h)], sem_g)

    for g0 in range(NB):
        emit(g0)

    def group_body(g, _):
        si = lax.rem(g, NI) * GR
        sb = lax.rem(g, NB) * GR
        so = lax.rem(g, NO) * (G * C)
        base = wid * (GROUPS_PER_W * G) + g * G

        pltpu.make_async_copy(feat_hbm.at[pl.ds(0, GR)],
                              buf_v.at[pl.ds(sb, GR)], sem_g).wait()

        @pl.when(g >= NO)
        def _():
            pltpu.make_async_copy(out_hbm.at[pl.ds(0, G * C)],
                                  ostage_v.at[pl.ds(so, G * C)], sem_o).wait()

        def o_body(o, _):
            def k_body(kk, accs):
                m = kk * L + o
                wv = plsc.load_gather(w_v, [lax.broadcast(si + m, (L,))])
                out = []
                for j in range(C // 32):
                    a, b = plsc.unpack(
                        plsc.bitcast(buf_v[sb + m, pl.ds(j * L, L)],
                                     jnp.bfloat16),
                        format=plsc.PackFormat.INTERLEAVED,
                        preferred_element_type=jnp.float32,
                    )
                    out.append(accs[j] + wv * a)
                    out.append(accs[j + C // 32] + wv * b)
                return tuple(out[0::2]) + tuple(out[1::2])

            accs = lax.fori_loop(
                0, SLOTS, k_body,
                tuple(jnp.zeros((L,), jnp.float32) for _ in range(C // L)),
            )
            pe = so + o * C + 2 * lax.iota(jnp.int32, L)
            for j in range(C // 32):
                plsc.store_scatter(ostage_v, [pe + 32 * j], accs[j])
                plsc.store_scatter(ostage_v, [pe + 32 * j + 1],
                                   accs[j + C // 32])
            return 0

        lax.fori_loop(0, G, o_body, 0)
        pltpu.async_copy(ostage_v.at[pl.ds(so, G * C)],
                         out_hbm.at[pl.ds(base * C, G * C)], sem_o)

        @pl.when(g + NB < GROUPS_PER_W)
        def _():
            emit(g + NB)

        return 0

    lax.fori_loop(0, GROUPS_PER_W, group_body, 0)
    for _ in range(NO):
        pltpu.make_async_copy(out_hbm.at[pl.ds(0, G * C)],
                              ostage_v.at[pl.ds(0, G * C)], sem_o).wait()


def kernel(input, rois):
    xb = input.astype(jnp.bfloat16)
    feat_bf = jnp.transpose(xb, (0, 2, 3, 1))
    feat_i32 = lax.bitcast_convert_type(
        feat_bf.reshape(N * H * W, C // 2, 2), jnp.int32)
    out_flat = _roi_pool_sc(feat_i32, rois.reshape(-1))
    return out_flat.reshape(R, PH, PW, C).transpose(0, 3, 1, 2)


# speedup vs baseline: 2.9336x; 2.2267x over previous
import functools

import numpy as np

import jax
import jax.numpy as jnp
from jax import lax
from jax.experimental import pallas as pl
from jax.experimental.pallas import tpu as pltpu
from jax.experimental.pallas import tpu_sc as plsc

N, C, H, W = 2, 256, 100, 152
R = 512
PH = PW = 7
SR = 2
SCALE = 0.125

NC, NS, L = 2, 16, 16
NW = NC * NS
OUT_ROWS = R * PH * PW
G = 16
GROUPS_PER_W = OUT_ROWS // (NW * G)
SLOTS = SR * SR * 4
GR = SLOTS * G

NB = 3
NI = 4
NO = 2

def _mesh():
    return plsc.VectorSubcoreMesh(
        core_axis_name="c", subcore_axis_name="s", num_cores=NC, num_subcores=NS
    )


@functools.partial(
    pl.kernel,
    out_type=jax.ShapeDtypeStruct((OUT_ROWS * C,), jnp.float32),
    mesh=_mesh(),
    compiler_params=pltpu.CompilerParams(needs_layout_passes=False),
    scratch_types=[
        pltpu.VMEM((R * 5,), jnp.float32),
        pltpu.VMEM((NI * GR,), jnp.int32),
        pltpu.VMEM((NI * GR,), jnp.float32),
        pltpu.VMEM((NB * GR, C // 2), jnp.int32),
        pltpu.VMEM((NO * G * C,), jnp.float32),
        pltpu.SemaphoreType.DMA,
        pltpu.SemaphoreType.DMA,
    ],
)
def _roi_pool_sc(feat_hbm, rois_hbm, out_hbm, rois_v, idx_v, w_v, buf_v,
                 ostage_v, sem_g, sem_o):
    wid = lax.axis_index("s") * NC + lax.axis_index("c")
    pltpu.sync_copy(rois_hbm, rois_v)

    def emit(g):
        si = lax.rem(g, NI) * GR
        sb = lax.rem(g, NB) * GR
        base = wid * (GROUPS_PER_W * G) + g * G
        orv = base + lax.iota(jnp.int32, L)
        r = lax.div(orv, PH * PW)
        rem = lax.rem(orv, PH * PW)
        ph = lax.div(rem, PW)
        pw = lax.rem(rem, PW)

        r5 = r * 5
        col = lambda c: plsc.load_gather(rois_v, [r5 + c])
        b_i = col(0).astype(jnp.int32)
        x1 = col(1) * SCALE - 0.5
        y1 = col(2) * SCALE - 0.5
        x2 = col(3) * SCALE - 0.5
        y2 = col(4) * SCALE - 0.5
        bw = jnp.maximum(x2 - x1, 1.0) * (1.0 / PW)
        bh = jnp.maximum(y2 - y1, 1.0) * (1.0 / PH)
        base_row = b_i * (H * W)
        ph_f = ph.astype(jnp.float32)
        pw_f = pw.astype(jnp.float32)

        wy, ry = [], []
        for s in range(SR):
            ys = y1 + (ph_f + (0.5 + s) / SR) * bh
            vy = jnp.where((ys > -1.0) & (ys < float(H)), 0.5, 0.0)
            yc = jnp.clip(ys, 0.0, float(H - 1))
            y0i = yc.astype(jnp.int32)
            ly = yc - y0i.astype(jnp.float32)
            wy.append([(1.0 - ly) * vy, ly * vy])
            ry.append([y0i * W, jnp.minimum(y0i + 1, H - 1) * W])
        wx, rx = [], []
        for t in range(SR):
            xs = x1 + (pw_f + (0.5 + t) / SR) * bw
            vx = jnp.where((xs > -1.0) & (xs < float(W)), 0.5, 0.0)
            xc = jnp.clip(xs, 0.0, float(W - 1))
            x0i = xc.astype(jnp.int32)
            lx = xc - x0i.astype(jnp.float32)
            wx.append([(1.0 - lx) * vx, lx * vx])
            rx.append([x0i, jnp.minimum(x0i + 1, W - 1)])

        k = 0
        for s in range(SR):
            for t in range(SR):
                for i in range(2):
                    for j in range(2):
                        idx_v[pl.ds(si + k * L, L)] = (
                            base_row + ry[s][i] + rx[t][j])
                        w_v[pl.ds(si + k * L, L)] = wy[s][i] * wx[t][j]
                        k += 1

        h = GR // 2
        pltpu.async_copy(feat_hbm.at[idx_v.at[pl.ds(si, h)]],
                         buf_v.at[pl.ds(sb, h)], sem_g)
        pltpu.async_copy(feat_hbm.at[idx_v.at[pl.ds(si + h, h)]],
                         buf_v.at[pl.ds(sb + h, h)], sem_g)

    for g0 in range(NB):
        emit(g0)

    def group_body(g, _):
        si = lax.rem(g, NI) * GR
        sb = lax.rem(g, NB) * GR
        so = lax.rem(g, NO) * (G * C)
        base = wid * (GROUPS_PER_W * G) + g * G

        pltpu.make_async_copy(feat_hbm.at[pl.ds(0, GR)],
                              buf_v.at[pl.ds(sb, GR)], sem_g).wait()

        @pl.when(g >= NO)
        def _():
            pltpu.make_async_copy(out_hbm.at[pl.ds(0, G * C)],
                                  ostage_v.at[pl.ds(so, G * C)], sem_o).wait()

        def o_body(o, _):
            def k_body(kk, accs):
                m = kk * L + o
                wv = plsc.load_gather(w_v, [lax.broadcast(si + m, (L,))])
                out = []
                for j in range(C // 32):
                    a, b = plsc.unpack(
                        plsc.bitcast(buf_v[sb + m, pl.ds(j * L, L)],
                                     jnp.bfloat16),
                        format=plsc.PackFormat.INTERLEAVED,
                        preferred_element_type=jnp.float32,
                    )
                    out.append(accs[j] + wv * a)
                    out.append(accs[j + C // 32] + wv * b)
                return tuple(out[0::2]) + tuple(out[1::2])

            accs = lax.fori_loop(
                0, SLOTS, k_body,
                tuple(jnp.zeros((L,), jnp.float32) for _ in range(C // L)),
            )
            for j in range(C // 32):
                ostage_v[pl.ds(so + o * C + L * j, L)] = accs[j]
                ostage_v[pl.ds(so + o * C + C // 2 + L * j, L)] = (
                    accs[j + C // 32])
            return 0

        lax.fori_loop(0, G, o_body, 0)
        pltpu.async_copy(ostage_v.at[pl.ds(so, G * C)],
                         out_hbm.at[pl.ds(base * C, G * C)], sem_o)

        @pl.when(g + NB < GROUPS_PER_W)
        def _():
            emit(g + NB)

        return 0

    lax.fori_loop(0, GROUPS_PER_W, group_body, 0)
    for _ in range(NO):
        pltpu.make_async_copy(out_hbm.at[pl.ds(0, G * C)],
                              ostage_v.at[pl.ds(0, G * C)], sem_o).wait()


def kernel(input, rois):
    xb = input.astype(jnp.bfloat16)
    lo = lax.bitcast_convert_type(xb[:, :C // 2], jnp.uint16).astype(jnp.uint32)
    hi = lax.bitcast_convert_type(xb[:, C // 2:], jnp.uint16).astype(jnp.uint32)
    w = lo | (hi << 16)
    feat_i32 = lax.bitcast_convert_type(
        jnp.transpose(w, (0, 2, 3, 1)).reshape(N * H * W, C // 2), jnp.int32)
    out_flat = _roi_pool_sc(feat_i32, rois.reshape(-1))
    return out_flat.reshape(R, PH, PW, C).transpose(0, 3, 1, 2)


# speedup vs baseline: 2.9483x; 1.0050x over previous
import functools

import numpy as np

import jax
import jax.numpy as jnp
from jax import lax
from jax.experimental import pallas as pl
from jax.experimental.pallas import tpu as pltpu
from jax.experimental.pallas import tpu_sc as plsc

N, C, H, W = 2, 256, 100, 152
R = 512
PH = PW = 7
SR = 2
SCALE = 0.125

NC, NS, L = 2, 16, 16
NW = NC * NS
OUT_ROWS = R * PH * PW
G = 16
GROUPS_PER_W = OUT_ROWS // (NW * G)
SLOTS = SR * SR * 4
GR = SLOTS * G

NB = 3
NI = 4
NO = 2

def _mesh():
    return plsc.VectorSubcoreMesh(
        core_axis_name="c", subcore_axis_name="s", num_cores=NC, num_subcores=NS
    )


@functools.partial(
    pl.kernel,
    out_type=jax.ShapeDtypeStruct((OUT_ROWS * C,), jnp.float32),
    mesh=_mesh(),
    compiler_params=pltpu.CompilerParams(needs_layout_passes=False),
    scratch_types=[
        pltpu.VMEM((R * 5,), jnp.float32),
        pltpu.VMEM((NI * GR,), jnp.int32),
        pltpu.VMEM((NI * GR,), jnp.float32),
        pltpu.VMEM((NB * GR, C // 2), jnp.int32),
        pltpu.VMEM((NO * G * C,), jnp.float32),
        pltpu.SemaphoreType.DMA,
        pltpu.SemaphoreType.DMA,
    ],
)
def _roi_pool_sc(feat_hbm, rois_hbm, out_hbm, rois_v, idx_v, w_v, buf_v,
                 ostage_v, sem_g, sem_o):
    wid = lax.axis_index("s") * NC + lax.axis_index("c")
    pltpu.sync_copy(rois_hbm, rois_v)

    def emit(g):
        si = lax.rem(g, NI) * GR
        sb = lax.rem(g, NB) * GR
        base = wid * (GROUPS_PER_W * G) + g * G
        orv = base + lax.iota(jnp.int32, L)
        r = lax.div(orv, PH * PW)
        rem = lax.rem(orv, PH * PW)
        ph = lax.div(rem, PW)
        pw = lax.rem(rem, PW)

        r5 = r * 5
        col = lambda c: plsc.load_gather(rois_v, [r5 + c])
        b_i = col(0).astype(jnp.int32)
        x1 = col(1) * SCALE - 0.5
        y1 = col(2) * SCALE - 0.5
        x2 = col(3) * SCALE - 0.5
        y2 = col(4) * SCALE - 0.5
        bw = jnp.maximum(x2 - x1, 1.0) * (1.0 / PW)
        bh = jnp.maximum(y2 - y1, 1.0) * (1.0 / PH)
        base_row = b_i * (H * W)
        ph_f = ph.astype(jnp.float32)
        pw_f = pw.astype(jnp.float32)

        wy, ry = [], []
        for s in range(SR):
            ys = y1 + (ph_f + (0.5 + s) / SR) * bh
            vy = jnp.where((ys > -1.0) & (ys < float(H)), 0.5, 0.0)
            yc = jnp.clip(ys, 0.0, float(H - 1))
            y0i = yc.astype(jnp.int32)
            ly = yc - y0i.astype(jnp.float32)
            wy.append([(1.0 - ly) * vy, ly * vy])
            ry.append([y0i * W, jnp.minimum(y0i + 1, H - 1) * W])
        wx, rx = [], []
        for t in range(SR):
            xs = x1 + (pw_f + (0.5 + t) / SR) * bw
            vx = jnp.where((xs > -1.0) & (xs < float(W)), 0.5, 0.0)
            xc = jnp.clip(xs, 0.0, float(W - 1))
            x0i = xc.astype(jnp.int32)
            lx = xc - x0i.astype(jnp.float32)
            wx.append([(1.0 - lx) * vx, lx * vx])
            rx.append([x0i, jnp.minimum(x0i + 1, W - 1)])

        k = 0
        for s in range(SR):
            for t in range(SR):
                for i in range(2):
                    for j in range(2):
                        idx_v[pl.ds(si + k * L, L)] = (
                            base_row + ry[s][i] + rx[t][j])
                        w_v[pl.ds(si + k * L, L)] = wy[s][i] * wx[t][j]
                        k += 1

        h = GR // 2
        pltpu.async_copy(feat_hbm.at[idx_v.at[pl.ds(si, h)]],
                         buf_v.at[pl.ds(sb, h)], sem_g)
        pltpu.async_copy(feat_hbm.at[idx_v.at[pl.ds(si + h, h)]],
                         buf_v.at[pl.ds(sb + h, h)], sem_g)

    for g0 in range(NB):
        emit(g0)

    def group_body(g, _):
        si = lax.rem(g, NI) * GR
        sb = lax.rem(g, NB) * GR
        so = lax.rem(g, NO) * (G * C)
        base = wid * (GROUPS_PER_W * G) + g * G

        pltpu.make_async_copy(feat_hbm.at[pl.ds(0, GR)],
                              buf_v.at[pl.ds(sb, GR)], sem_g).wait()

        @pl.when(g >= NO)
        def _():
            pltpu.make_async_copy(out_hbm.at[pl.ds(0, G * C)],
                                  ostage_v.at[pl.ds(so, G * C)], sem_o).wait()

        def o_body(o, _):
            accs = [jnp.zeros((L,), jnp.float32) for _ in range(C // L)]
            for kk in range(SLOTS):
                m = kk * L + o
                wv = plsc.load_gather(w_v, [lax.broadcast(si + m, (L,))])
                for j in range(C // 32):
                    a, b = plsc.unpack(
                        plsc.bitcast(buf_v[sb + m, pl.ds(j * L, L)],
                                     jnp.bfloat16),
                        format=plsc.PackFormat.INTERLEAVED,
                        preferred_element_type=jnp.float32,
                    )
                    accs[j] = accs[j] + wv * a
                    accs[j + C // 32] = accs[j + C // 32] + wv * b
            for j in range(C // 32):
                ostage_v[pl.ds(so + o * C + L * j, L)] = accs[j]
                ostage_v[pl.ds(so + o * C + C // 2 + L * j, L)] = (
                    accs[j + C // 32])
            return 0

        lax.fori_loop(0, G, o_body, 0)
        pltpu.async_copy(ostage_v.at[pl.ds(so, G * C)],
                         out_hbm.at[pl.ds(base * C, G * C)], sem_o)

        @pl.when(g + NB < GROUPS_PER_W)
        def _():
            emit(g + NB)

        return 0

    lax.fori_loop(0, GROUPS_PER_W, group_body, 0)
    for _ in range(NO):
        pltpu.make_async_copy(out_hbm.at[pl.ds(0, G * C)],
                              ostage_v.at[pl.ds(0, G * C)], sem_o).wait()


def kernel(input, rois):
    xb = input.astype(jnp.bfloat16)
    lo = lax.bitcast_convert_type(xb[:, :C // 2], jnp.uint16).astype(jnp.uint32)
    hi = lax.bitcast_convert_type(xb[:, C // 2:], jnp.uint16).astype(jnp.uint32)
    w = lo | (hi << 16)
    feat_i32 = lax.bitcast_convert_type(
        jnp.transpose(w, (0, 2, 3, 1)).reshape(N * H * W, C // 2), jnp.int32)
    out_flat = _roi_pool_sc(feat_i32, rois.reshape(-1))
    return out_flat.reshape(R, PH, PW, C).transpose(0, 3, 1, 2)


# speedup vs baseline: 3.8129x; 1.2933x over previous
import functools

import numpy as np

import jax
import jax.numpy as jnp
from jax import lax
from jax.experimental import pallas as pl
from jax.experimental.pallas import tpu as pltpu
from jax.experimental.pallas import tpu_sc as plsc

N, C, H, W = 2, 256, 100, 152
R = 512
PH = PW = 7
SR = 2
SCALE = 0.125

NC, NS, L = 2, 16, 16
NW = NC * NS
OUT_ROWS = R * PH * PW
G = 16
GROUPS_PER_W = OUT_ROWS // (NW * G)
SLOTS = SR * SR * 4
GR = SLOTS * G

NB = 3
NI = 4
NO = 2

def _mesh():
    return plsc.VectorSubcoreMesh(
        core_axis_name="c", subcore_axis_name="s", num_cores=NC, num_subcores=NS
    )


@functools.partial(
    pl.kernel,
    out_type=jax.ShapeDtypeStruct((OUT_ROWS, C), jnp.float32),
    mesh=_mesh(),
    compiler_params=pltpu.CompilerParams(needs_layout_passes=False),
    scratch_types=[
        pltpu.VMEM((R * 5,), jnp.float32),
        pltpu.VMEM((NI * GR,), jnp.int32),
        pltpu.VMEM((NI * GR,), jnp.float32),
        pltpu.VMEM((NB * GR, C // 2), jnp.int32),
        pltpu.VMEM((NO * G, C), jnp.float32),
        pltpu.VMEM((NO, L), jnp.int32),
        pltpu.SemaphoreType.DMA,
        pltpu.SemaphoreType.DMA,
    ],
)
def _roi_pool_sc(feat_hbm, rois_hbm, out_hbm, rois_v, idx_v, w_v, buf_v,
                 ostage_v, oidx_v, sem_g, sem_o):
    wid = lax.axis_index("s") * NC + lax.axis_index("c")
    pltpu.sync_copy(rois_hbm, rois_v)

    def emit(g):
        si = lax.rem(g, NI) * GR
        sb = lax.rem(g, NB) * GR
        base = wid * (GROUPS_PER_W * G) + g * G
        orv = base + lax.iota(jnp.int32, L)
        r = lax.div(orv, PH * PW)
        rem = lax.rem(orv, PH * PW)
        ph = lax.div(rem, PW)
        pw = lax.rem(rem, PW)

        r5 = r * 5
        col = lambda c: plsc.load_gather(rois_v, [r5 + c])
        b_i = col(0).astype(jnp.int32)
        x1 = col(1) * SCALE - 0.5
        y1 = col(2) * SCALE - 0.5
        x2 = col(3) * SCALE - 0.5
        y2 = col(4) * SCALE - 0.5
        bw = jnp.maximum(x2 - x1, 1.0) * (1.0 / PW)
        bh = jnp.maximum(y2 - y1, 1.0) * (1.0 / PH)
        base_row = b_i * (H * W)
        ph_f = ph.astype(jnp.float32)
        pw_f = pw.astype(jnp.float32)

        wy, ry = [], []
        for s in range(SR):
            ys = y1 + (ph_f + (0.5 + s) / SR) * bh
            vy = jnp.where((ys > -1.0) & (ys < float(H)), 0.5, 0.0)
            yc = jnp.clip(ys, 0.0, float(H - 1))
            y0i = yc.astype(jnp.int32)
            ly = yc - y0i.astype(jnp.float32)
            wy.append([(1.0 - ly) * vy, ly * vy])
            ry.append([y0i * W, jnp.minimum(y0i + 1, H - 1) * W])
        wx, rx = [], []
        for t in range(SR):
            xs = x1 + (pw_f + (0.5 + t) / SR) * bw
            vx = jnp.where((xs > -1.0) & (xs < float(W)), 0.5, 0.0)
            xc = jnp.clip(xs, 0.0, float(W - 1))
            x0i = xc.astype(jnp.int32)
            lx = xc - x0i.astype(jnp.float32)
            wx.append([(1.0 - lx) * vx, lx * vx])
            rx.append([x0i, jnp.minimum(x0i + 1, W - 1)])

        k = 0
        for s in range(SR):
            for t in range(SR):
                for i in range(2):
                    for j in range(2):
                        idx_v[pl.ds(si + k * L, L)] = (
                            base_row + ry[s][i] + rx[t][j])
                        w_v[pl.ds(si + k * L, L)] = wy[s][i] * wx[t][j]
                        k += 1

        h = GR // 2
        pltpu.async_copy(feat_hbm.at[idx_v.at[pl.ds(si, h)]],
                         buf_v.at[pl.ds(sb, h)], sem_g)
        pltpu.async_copy(feat_hbm.at[idx_v.at[pl.ds(si + h, h)]],
                         buf_v.at[pl.ds(sb + h, h)], sem_g)

    for g0 in range(NB):
        emit(g0)

    def group_body(g, _):
        si = lax.rem(g, NI) * GR
        sb = lax.rem(g, NB) * GR
        soslot = lax.rem(g, NO)
        so = soslot * G
        base = wid * (GROUPS_PER_W * G) + g * G

        pltpu.make_async_copy(feat_hbm.at[pl.ds(0, GR)],
                              buf_v.at[pl.ds(sb, GR)], sem_g).wait()

        @pl.when(g >= NO)
        def _():
            pltpu.make_async_copy(out_hbm.at[pl.ds(0, G)],
                                  ostage_v.at[pl.ds(so, G)], sem_o).wait()

        def o_body(o, _):
            accs = [jnp.zeros((L,), jnp.float32) for _ in range(C // L)]
            for kk in range(SLOTS):
                m = kk * L + o
                wv = plsc.load_gather(w_v, [lax.broadcast(si + m, (L,))])
                for j in range(C // 32):
                    a, b = plsc.unpack(
                        plsc.bitcast(buf_v[sb + m, pl.ds(j * L, L)],
                                     jnp.bfloat16),
                        format=plsc.PackFormat.INTERLEAVED,
                        preferred_element_type=jnp.float32,
                    )
                    accs[j] = accs[j] + wv * a
                    accs[j + C // 32] = accs[j + C // 32] + wv * b
            for j in range(C // 32):
                ostage_v[so + o, pl.ds(L * j, L)] = accs[j]
                ostage_v[so + o, pl.ds(C // 2 + L * j, L)] = (
                    accs[j + C // 32])
            return 0

        lax.fori_loop(0, G, o_body, 0)
        orv = base + lax.iota(jnp.int32, L)
        r = lax.div(orv, PH * PW)
        oidx_v[soslot] = lax.rem(orv, PH * PW) * R + r
        pltpu.async_copy(ostage_v.at[pl.ds(so, G)],
                         out_hbm.at[oidx_v.at[soslot]], sem_o)

        @pl.when(g + NB < GROUPS_PER_W)
        def _():
            emit(g + NB)

        return 0

    lax.fori_loop(0, GROUPS_PER_W, group_body, 0)
    for _ in range(NO):
        pltpu.make_async_copy(out_hbm.at[pl.ds(0, G)],
                              ostage_v.at[pl.ds(0, G)], sem_o).wait()


def kernel(input, rois):
    xb = input.astype(jnp.bfloat16)
    lo = lax.bitcast_convert_type(xb[:, :C // 2], jnp.uint16).astype(jnp.uint32)
    hi = lax.bitcast_convert_type(xb[:, C // 2:], jnp.uint16).astype(jnp.uint32)
    w = lo | (hi << 16)
    feat_i32 = lax.bitcast_convert_type(
        jnp.transpose(w, (0, 2, 3, 1)).reshape(N * H * W, C // 2), jnp.int32)
    out_rows = _roi_pool_sc(feat_i32, rois.reshape(-1))
    return out_rows.reshape(PH, PW, R, C).transpose(2, 3, 0, 1)
